# baseline (speedup 1.0000x reference)
import functools

import jax
import jax.numpy as jnp
from jax import lax
from jax.experimental import pallas as pl
from jax.experimental.pallas import tpu as pltpu
from jax.experimental.pallas import tpu_sc as plsc

LANES = 16
NUM_WORKERS = 32


def _build_kernel(tok, hidden, seq):
    tpw = tok // NUM_WORKERS
    idx_rows = tpw // 128
    jh = hidden // LANES
    wpb = seq // tpw

    mesh = plsc.VectorSubcoreMesh(core_axis_name="c", subcore_axis_name="s")

    @functools.partial(
        pl.kernel,
        mesh=mesh,
        out_type=jax.ShapeDtypeStruct((tok, hidden), jnp.float32),
        scratch_types=[
            pltpu.VMEM((idx_rows, 128), jnp.int32),
            pltpu.VMEM((idx_rows, 128), jnp.int32),
            pltpu.VMEM((tpw, hidden), jnp.float32),
            pltpu.VMEM((tpw, hidden), jnp.float32),
            pltpu.VMEM((tpw, hidden), jnp.float32),
            pltpu.VMEM((hidden,), jnp.float32),
            pltpu.VMEM((hidden,), jnp.float32),
            pltpu.SemaphoreType.DMA,
        ],
    )
    def embed_ln(ids_hbm, tt_hbm, word_hbm, pos_hbm, type_hbm, gamma_hbm,
                 beta_hbm, out_hbm, idx_v, tti_v, we_v, pe_v, te_v, g_v, b_v,
                 sem):
        wid = lax.axis_index("s") * 2 + lax.axis_index("c")
        base = wid * tpw

        pltpu.sync_copy(ids_hbm.at[pl.ds(wid * idx_rows, idx_rows)], idx_v)
        pltpu.sync_copy(tt_hbm.at[pl.ds(wid * idx_rows, idx_rows)], tti_v)

        copies = []
        for r in range(idx_rows):
            dst = pl.ds(r * 128, 128)
            copies.append(
                pltpu.async_copy(word_hbm.at[idx_v.at[r]], we_v.at[dst], sem))
            copies.append(
                pltpu.async_copy(type_hbm.at[tti_v.at[r]], te_v.at[dst], sem))

        pltpu.sync_copy(pos_hbm.at[pl.ds((wid % wpb) * tpw, tpw)], pe_v)
        pltpu.sync_copy(gamma_hbm, g_v)
        pltpu.sync_copy(beta_hbm, b_v)
        for cp in copies:
            cp.wait()

        g = [g_v[pl.ds(LANES * j, LANES)] for j in range(jh)]
        b = [b_v[pl.ds(LANES * j, LANES)] for j in range(jh)]
        lane = lax.iota(jnp.int32, LANES)
        perms = [jnp.bitwise_xor(lane, k)[:, None] for k in (8, 4, 2, 1)]
        dnums = lax.GatherDimensionNumbers(
            offset_dims=(), collapsed_slice_dims=(0,), start_index_map=(0,))

        def hsum(v):
            for p in perms:
                v = v + lax.gather(v, p, dnums, slice_sizes=(1,),
                                   mode=lax.GatherScatterMode.PROMISE_IN_BOUNDS)
            return v

        def token(t, carry):
            acc = jnp.zeros((LANES,), jnp.float32)
            acc2 = jnp.zeros((LANES,), jnp.float32)
            e = []
            for j in range(jh):
                sl = pl.ds(LANES * j, LANES)
                v = we_v[t, sl] + pe_v[t, sl] + te_v[t, sl]
                e.append(v)
                acc = acc + v
                acc2 = acc2 + v * v
            mean = hsum(acc) * (1.0 / hidden)
            var = hsum(acc2) * (1.0 / hidden) - mean * mean
            x = var + 1e-12
            i = lax.bitcast_convert_type(x, jnp.int32)
            i = 0x5F3759DF - lax.shift_right_arithmetic(i, 1)
            y = lax.bitcast_convert_type(i, jnp.float32)
            for _ in range(3):
                y = y * (1.5 - 0.5 * x * y * y)
            for j in range(jh):
                we_v[t, pl.ds(LANES * j, LANES)] = (e[j] - mean) * y * g[j] + b[j]
            return carry

        lax.fori_loop(0, tpw, token, 0)
        pltpu.sync_copy(we_v, out_hbm.at[pl.ds(base, tpw)])

    return embed_ln


def kernel(input_ids, token_type_ids, word_table, pos_table, type_table,
           ln_gamma, ln_beta):
    batch, seq = input_ids.shape
    hidden = word_table.shape[1]
    tok = batch * seq
    ids = input_ids.astype(jnp.int32).reshape(tok // 128, 128)
    tts = token_type_ids.astype(jnp.int32).reshape(tok // 128, 128)
    fn = _build_kernel(tok, hidden, seq)
    out = fn(ids, tts, word_table, pos_table, type_table, ln_gamma, ln_beta)
    return out.reshape(batch, seq, hidden)


# speedup vs baseline: 5.0222x; 5.0222x over previous
import functools

import jax
import jax.numpy as jnp
from jax import lax
from jax.experimental import pallas as pl
from jax.experimental.pallas import tpu as pltpu
from jax.experimental.pallas import tpu_sc as plsc

LANES = 16
NUM_WORKERS = 32


def _build_kernel(tok, hidden, seq):
    tpw = tok // NUM_WORKERS
    idx_rows = tpw // 128
    jh = hidden // LANES
    wpb = seq // tpw

    mesh = plsc.VectorSubcoreMesh(core_axis_name="c", subcore_axis_name="s")

    @functools.partial(
        pl.kernel,
        mesh=mesh,
        out_type=jax.ShapeDtypeStruct((tok, hidden), jnp.float32),
        scratch_types=[
            pltpu.VMEM((idx_rows, 128), jnp.int32),
            pltpu.VMEM((tpw,), jnp.int32),
            pltpu.VMEM((tpw, hidden), jnp.float32),
            pltpu.VMEM((tpw, hidden), jnp.float32),
            pltpu.VMEM((hidden,), jnp.float32),
            pltpu.VMEM((hidden,), jnp.float32),
            pltpu.VMEM((hidden,), jnp.float32),
            pltpu.SemaphoreType.DMA,
            pltpu.SemaphoreType.DMA,
        ],
    )
    def embed_ln(ids_hbm, tt_hbm, word_hbm, pos_hbm, drow_hbm, gamma_hbm,
                 beta_hbm, out_hbm, idx_v, tti_v, we_v, pe_v, d_v, g_v, b_v,
                 gsem, lsem):
        wid = lax.axis_index("s") * 2 + lax.axis_index("c")
        base = wid * tpw

        pltpu.sync_copy(ids_hbm.at[pl.ds(wid * idx_rows, idx_rows)], idx_v)
        copies = []
        for r in range(idx_rows):
            dst = pl.ds(r * 128, 128)
            copies.append(
                pltpu.async_copy(word_hbm.at[idx_v.at[r]], we_v.at[dst],
                                 gsem))

        copies.append(pltpu.async_copy(tt_hbm.at[pl.ds(base, tpw)], tti_v,
                                       lsem))
        copies.append(pltpu.async_copy(
            pos_hbm.at[pl.ds((wid % wpb) * tpw, tpw)], pe_v, lsem))
        copies.append(pltpu.async_copy(drow_hbm, d_v, lsem))
        copies.append(pltpu.async_copy(gamma_hbm, g_v, lsem))
        copies.append(pltpu.async_copy(beta_hbm, b_v, lsem))
        for cp in copies:
            cp.wait()

        g = [g_v[pl.ds(LANES * j, LANES)] for j in range(jh)]
        b = [b_v[pl.ds(LANES * j, LANES)] for j in range(jh)]
        d = [d_v[pl.ds(LANES * j, LANES)] for j in range(jh)]
        lane = lax.iota(jnp.int32, LANES)
        perms = [jnp.bitwise_xor(lane, k)[:, None] for k in (8, 4, 2, 1)]
        dnums = lax.GatherDimensionNumbers(
            offset_dims=(), collapsed_slice_dims=(0,), start_index_map=(0,))

        def hsum(v):
            for p in perms:
                v = v + lax.gather(v, p, dnums, slice_sizes=(1,),
                                   mode=lax.GatherScatterMode.PROMISE_IN_BOUNDS)
            return v

        def token(t, ttf):
            acc = jnp.zeros((LANES,), jnp.float32)
            acc2 = jnp.zeros((LANES,), jnp.float32)
            e = []
            for j in range(jh):
                sl = pl.ds(LANES * j, LANES)
                v = we_v[t, sl] + pe_v[t, sl] + ttf * d[j]
                e.append(v)
                acc = acc + v
                acc2 = acc2 + v * v
            mean = hsum(acc) * (1.0 / hidden)
            var = hsum(acc2) * (1.0 / hidden) - mean * mean
            x = var + 1e-12
            i = lax.bitcast_convert_type(x, jnp.int32)
            i = 0x5F3759DF - lax.shift_right_arithmetic(i, 1)
            y = lax.bitcast_convert_type(i, jnp.float32)
            for _ in range(2):
                y = y * (1.5 - 0.5 * x * y * y)
            for j in range(jh):
                we_v[t, pl.ds(LANES * j, LANES)] = (e[j] - mean) * y * g[j] + b[j]

        def group(gi, carry):
            t0 = gi * LANES
            ttf16 = tti_v[pl.ds(t0, LANES)].astype(jnp.float32)
            for tk in range(LANES):
                token(t0 + tk, ttf16[tk])
            return carry

        lax.fori_loop(0, tpw // LANES, group, 0)
        pltpu.sync_copy(we_v, out_hbm.at[pl.ds(base, tpw)])

    return embed_ln


def kernel(input_ids, token_type_ids, word_table, pos_table, type_table,
           ln_gamma, ln_beta):
    batch, seq = input_ids.shape
    hidden = word_table.shape[1]
    tok = batch * seq
    ids = input_ids.astype(jnp.int32).reshape(tok // 128, 128)
    tts = token_type_ids.astype(jnp.int32).reshape(tok)
    pos_eff = pos_table + type_table[0]
    drow = type_table[1] - type_table[0]
    fn = _build_kernel(tok, hidden, seq)
    out = fn(ids, tts, word_table, pos_eff, drow, ln_gamma, ln_beta)
    return out.reshape(batch, seq, hidden)
